# baseline (speedup 1.0000x reference)
import functools

import jax
import jax.numpy as jnp
from jax import lax
from jax.experimental import pallas as pl
from jax.experimental.pallas import tpu as pltpu
from jax.experimental.pallas import tpu_sc as plsc

DIM = 64
B = 16384
L2_REG = 1e-4
V = 1000000

NC = 2
NS = 16
L = 16
NW = NC * NS
BPW = B // NW

CAPB = 32
NBLK = 7936
SPW = NBLK * CAPB
SPARE = SPW - 1
FILLC = SPW // NS // 8
NWIN = 1968
WPT = 123
LASTW = 999424
TAIL_BASE = 999936


def _sc_bucket_sweep(tu, ti, uid, iid):
    mesh = plsc.VectorSubcoreMesh(core_axis_name="c", subcore_axis_name="s")

    @functools.partial(
        pl.kernel,
        mesh=mesh,
        compiler_params=pltpu.CompilerParams(needs_layout_passes=False),
        out_type=(
            jax.ShapeDtypeStruct((B + 8, 128), jnp.float32),
            jax.ShapeDtypeStruct((2 * B + 8, 128), jnp.float32),
        ),
        scratch_types=[
            pltpu.VMEM((2048,), jnp.int32),
            pltpu.VMEM((2048,), jnp.int32),
            pltpu.VMEM((16, 128), jnp.int32),
            pltpu.VMEM((2048,), jnp.int32),
            pltpu.VMEM((DIM, 512), jnp.float32),
            pltpu.VMEM((DIM, 512), jnp.float32),
            pltpu.VMEM((DIM, 64), jnp.float32),
            pltpu.VMEM((128, 128), jnp.float32),
            pltpu.VMEM((128,), jnp.int32),
            pltpu.VMEM((128,), jnp.int32),
            pltpu.VMEM((1, 128), jnp.int32),
            pltpu.SMEM((512,), jnp.int32),
            pltpu.VMEM_SHARED((SPW,), jnp.int32),
            pltpu.VMEM_SHARED((SPW,), jnp.int32),
            pltpu.SemaphoreType.DMA,
            pltpu.SemaphoreType.DMA,
            pltpu.SemaphoreType.DMA,
        ],
    )
    def k(tu_h, ti_h, uid_h, iid_h, ou, oi,
          idsv, posv, posv2, payv, wba, wbb, tblk, stage, idsl, dstl, dstv,
          cnts, ids_sp, dst_sp, sa, sb, ssc):
        core = lax.axis_index("c")
        sub = lax.axis_index("s")
        lanes = lax.iota(jnp.int32, L)

        def zero_counters():
            def zc(i, _):
                cnts[i] = 0
                return 0

            lax.fori_loop(0, 512, zc, 0)

        def prefill(dump):
            base = sub * (SPW // NS)

            def st(buf, val):
                vv = jnp.full((L,), val, jnp.int32)
                for g in range(128):
                    buf[pl.ds(g * L, L)] = vv

            st(idsv, -1)
            st(payv, dump)

            def cp(i, _):
                off = pl.multiple_of(base + i * FILLC, FILLC)
                pltpu.sync_copy(
                    idsv.at[pl.ds(0, FILLC)],
                    ids_sp.at[pl.ds(off, FILLC)],
                )
                pltpu.sync_copy(
                    payv.at[pl.ds(0, FILLC)],
                    dst_sp.at[pl.ds(off, FILLC)],
                )
                return 0

            lax.fori_loop(0, 8, cp, 0)

        def bucket(ids_h, n, dump):
            npt = n // NS
            base = pl.multiple_of(sub * npt, npt)
            pltpu.sync_copy(ids_h.at[pl.ds(base, npt)], idsv.at[pl.ds(0, npt)])

            def grp(g, _):
                gb = pl.multiple_of(g * L, L)
                iv = idsv[pl.ds(gb, L)]
                pv = jnp.zeros((L,), jnp.int32)
                for r in range(L):
                    bb = iv[r] >> 7
                    owner = bb >> 9
                    slot = plsc.fetch_and_add(
                        cnts.at[bb & 511], 1, subcore_id=owner
                    )
                    gpos = jnp.where(
                        slot < CAPB, bb * CAPB + slot, SPARE
                    )
                    pv = jnp.where(lanes == r, gpos, pv)
                posv[pl.ds(gb, L)] = pv
                payv[pl.ds(gb, L)] = lanes + (base + gb)
                return 0

            lax.fori_loop(0, npt // L, grp, 0)
            for row in range(npt // 128):
                for gg in range(8):
                    posv2[row, pl.ds(gg * L, L)] = posv[
                        pl.ds(row * 128 + gg * L, L)
                    ]
            for m in range(npt // 128):
                pltpu.sync_copy(
                    idsv.at[pl.ds(m * 128, 128)], ids_sp.at[posv2.at[m]]
                )
                pltpu.sync_copy(
                    payv.at[pl.ds(m * 128, 128)], dst_sp.at[posv2.at[m]]
                )

        def extract_block(buf, wstart, joff32, klocal):
            iv0 = idsl[pl.ds(joff32, L)]
            iv1 = idsl[pl.ds(joff32 + L, L)]
            c0 = plsc.all_reduce_population_count(
                iv0 >= jnp.zeros((L,), jnp.int32)
            )
            c1 = plsc.all_reduce_population_count(
                iv1 >= jnp.zeros((L,), jnp.int32)
            )
            cnt = c0[0] + c1[0]
            m0 = lanes == 0

            def hit(kk, kl):
                i_id = idsl[pl.ds(joff32 + kk, L)][0]
                dd = dstl[pl.ds(joff32 + kk, L)][0]
                col = i_id - wstart
                cv = jnp.full((L,), 0, jnp.int32) + col
                for k4 in range(4):
                    v = plsc.load_gather(buf, [lanes + L * k4, cv])
                    stage[kl, pl.ds(L * k4, L)] = v
                plsc.store_scatter(
                    dstv, [jnp.zeros((L,), jnp.int32),
                           jnp.full((L,), 0, jnp.int32) + kl],
                    jnp.full((L,), 0, jnp.int32) + dd,
                    mask=m0,
                )
                return kl + 1

            return lax.fori_loop(0, cnt, hit, klocal)

        def do_window(t_h, out_h, w, buf, dump):
            wstart = pl.multiple_of(jnp.minimum(w * 512, LASTW), 512)
            b0w = pl.multiple_of(lax.shift_right_logical(wstart, 2), 128)
            pltpu.sync_copy(ids_sp.at[pl.ds(b0w, 128)], idsl)
            pltpu.sync_copy(dst_sp.at[pl.ds(b0w, 128)], dstl)
            dmp = jnp.full((L,), dump, jnp.int32)
            for g in range(8):
                dstv[0, pl.ds(g * L, L)] = dmp
            kl = 0
            for jb in range(4):
                kl = extract_block(buf, wstart, jb * 32, kl)
            pltpu.async_copy(
                stage.at[pl.ds(0, 128), :], out_h.at[dstv.at[0]], ssc
            ).wait()

        def issue(t_h, w, buf, sem):
            ws = pl.multiple_of(jnp.minimum(w * 512, LASTW), 512)
            for a in range(8):
                pltpu.async_copy(
                    t_h.at[pl.ds(8 * a, 8), pl.ds(ws, 512)],
                    buf.at[pl.ds(8 * a, 8), :],
                    sem,
                )

        def wait_all(t_h, buf, sem):
            for a in range(8):
                pltpu.make_async_copy(
                    t_h.at[pl.ds(0, 8), pl.ds(0, 512)],
                    buf.at[pl.ds(8 * a, 8), :],
                    sem,
                ).wait()

        def sweep(t_h, out_h, dump):
            wbase = sub
            issue(t_h, wbase, wba, sa)
            issue(t_h, wbase + NS, wbb, sb)

            def it(qq, _):
                w = wbase + 2 * qq * NS
                wait_all(t_h, wba, sa)
                do_window(t_h, out_h, w, wba, dump)
                issue(t_h, w + 2 * NS, wba, sa)
                wait_all(t_h, wbb, sb)
                do_window(t_h, out_h, w + NS, wbb, dump)
                issue(t_h, w + 3 * NS, wbb, sb)
                return 0

            lax.fori_loop(0, 62, it, 0)
            wait_all(t_h, wba, sa)
            wait_all(t_h, wbb, sb)

        def tail(t_h, out_h, dump):
            pltpu.async_copy(
                t_h.at[:, pl.ds(TAIL_BASE, 64)], tblk, sa
            ).wait()
            b0w = 7812 * CAPB
            pltpu.sync_copy(ids_sp.at[pl.ds(b0w, 128)], idsl)
            pltpu.sync_copy(dst_sp.at[pl.ds(b0w, 128)], dstl)
            dmp = jnp.full((L,), dump, jnp.int32)
            for g in range(8):
                dstv[0, pl.ds(g * L, L)] = dmp
            extract_block(tblk, TAIL_BASE, 0, 0)
            pltpu.async_copy(
                stage.at[pl.ds(0, 128), :], out_h.at[dstv.at[0]], ssc
            ).wait()

        def role(t_h, ids_h, out_h, n):
            zero_counters()
            prefill(n)
            plsc.subcore_barrier()
            bucket(ids_h, n, n)
            plsc.subcore_barrier()
            sweep(t_h, out_h, n)

            @pl.when(sub == 15)
            def _():
                tail(t_h, out_h, n)

        @pl.when(core == 0)
        def _():
            role(tu_h, uid_h, ou, B)

        @pl.when(core == 1)
        def _():
            role(ti_h, iid_h, oi, 2 * B)

    return k(tu, ti, uid, iid)


def _sc_scores(ou, oi):
    mesh = plsc.VectorSubcoreMesh(core_axis_name="c", subcore_axis_name="s")
    HALF = BPW // 2
    HG = HALF // L

    @functools.partial(
        pl.kernel,
        mesh=mesh,
        compiler_params=pltpu.CompilerParams(needs_layout_passes=False),
        out_type=(
            jax.ShapeDtypeStruct((B,), jnp.float32),
            jax.ShapeDtypeStruct((B,), jnp.float32),
            jax.ShapeDtypeStruct((NW, L), jnp.float32),
        ),
        scratch_types=[
            pltpu.VMEM((HALF, 128), jnp.float32),
            pltpu.VMEM((HALF, 128), jnp.float32),
            pltpu.VMEM((HALF, 128), jnp.float32),
            pltpu.VMEM((BPW,), jnp.float32),
            pltpu.VMEM((BPW,), jnp.float32),
            pltpu.VMEM((L,), jnp.float32),
            pltpu.VMEM((L * L,), jnp.float32),
            pltpu.VMEM((L * L,), jnp.float32),
            pltpu.SemaphoreType.DMA,
            pltpu.SemaphoreType.DMA,
            pltpu.SemaphoreType.DMA,
        ],
    )
    def k(ou_h, oi_h, pos_out, neg_out, sq_out,
          ubuf, pbuf, nbuf, psc, nsc, sqv, tpm, tnm, su, sp, sn):
        wid = lax.axis_index("s") * NC + lax.axis_index("c")
        base = wid * BPW
        lanes = lax.iota(jnp.int32, L)

        def half(h, sq):
            hb = pl.multiple_of(h * HALF, HALF)
            cu = pltpu.async_copy(ou_h.at[pl.ds(base + hb, HALF), :], ubuf, su)
            cp = pltpu.async_copy(oi_h.at[pl.ds(base + hb, HALF), :], pbuf, sp)
            cn = pltpu.async_copy(
                oi_h.at[pl.ds(B + base + hb, HALF), :], nbuf, sn)
            cu.wait()
            cp.wait()
            cn.wait()

            def group(g, sq):
                gbase = pl.multiple_of(g * L, L)
                for r in range(L):
                    tp = jnp.zeros((L,), jnp.float32)
                    tn = jnp.zeros((L,), jnp.float32)
                    sr = jnp.zeros((L,), jnp.float32)
                    for kk in range(DIM // L):
                        u = ubuf[gbase + r, pl.ds(kk * L, L)]
                        p = pbuf[gbase + r, pl.ds(kk * L, L)]
                        n = nbuf[gbase + r, pl.ds(kk * L, L)]
                        tp = tp + u * p
                        tn = tn + u * n
                        sr = sr + (u * u + (p * p + n * n))
                    sq = sq + sr
                    colidx = lanes * L + r
                    plsc.store_scatter(tpm, [colidx], tp)
                    plsc.store_scatter(tnm, [colidx], tn)
                pos_v = jnp.zeros((L,), jnp.float32)
                neg_v = jnp.zeros((L,), jnp.float32)
                for l in range(L):
                    pos_v = pos_v + tpm[pl.ds(l * L, L)]
                    neg_v = neg_v + tnm[pl.ds(l * L, L)]
                psc[pl.ds(hb + gbase, L)] = pos_v
                nsc[pl.ds(hb + gbase, L)] = neg_v
                return sq

            return lax.fori_loop(0, HG, group, sq)

        sq = lax.fori_loop(0, 2, half, jnp.zeros((L,), jnp.float32))
        sqv[...] = sq
        pltpu.sync_copy(psc, pos_out.at[pl.ds(base, BPW)])
        pltpu.sync_copy(nsc, neg_out.at[pl.ds(base, BPW)])
        pltpu.sync_copy(sqv, sq_out.at[wid])

    return k(ou, oi)


def _tc_finalize(pos2, neg2, sq2):
    def body(p_ref, n_ref, s_ref, bpr_ref, auc_ref, reg_ref):
        p = p_ref[...]
        n = n_ref[...]
        d = n - p
        sp = jnp.maximum(d, 0.0) + jnp.log(1.0 + jnp.exp(-jnp.abs(d)))
        bpr_ref[0, 0] = jnp.sum(sp) * (1.0 / B)
        auc_ref[0, 0] = jnp.sum((p > n).astype(jnp.float32)) * (1.0 / B)
        reg_ref[0, 0] = (0.5 * L2_REG / B) * jnp.sum(s_ref[...])

    return pl.pallas_call(
        body,
        out_shape=(
            jax.ShapeDtypeStruct((1, 1), jnp.float32),
            jax.ShapeDtypeStruct((1, 1), jnp.float32),
            jax.ShapeDtypeStruct((1, 1), jnp.float32),
        ),
        out_specs=(
            pl.BlockSpec(memory_space=pltpu.SMEM),
            pl.BlockSpec(memory_space=pltpu.SMEM),
            pl.BlockSpec(memory_space=pltpu.SMEM),
        ),
    )(pos2, neg2, sq2)


def kernel(user_table, item_table, users_id, pos_items_id, neg_items_id):
    uid = users_id.astype(jnp.int32)
    pid = pos_items_id.astype(jnp.int32)
    nid = neg_items_id.astype(jnp.int32)
    iid = jnp.concatenate([pid, nid])
    ou, oi = _sc_bucket_sweep(user_table.T, item_table.T, uid, iid)
    pos_s, neg_s, sq = _sc_scores(ou, oi)
    bpr, auc, reg = _tc_finalize(
        pos_s.reshape(128, 128), neg_s.reshape(128, 128), sq.reshape(4, 128)
    )
    return (bpr[0, 0], auc[0, 0], reg[0, 0])
